# Initial kernel scaffold; baseline (speedup 1.0000x reference)
import jax
import jax.numpy as jnp
from jax import lax
from jax.experimental import pallas as pl


def _body(x3_ref, der_ref, out_ref):
    j3 = lax.broadcasted_iota(jnp.int32, (x3_ref.shape[1], 3), 0) % 3
    k3 = lax.broadcasted_iota(jnp.int32, (x3_ref.shape[1], 3), 1)
    w = jnp.where(j3 == k3, x3_ref[0][:, None], 0.0)
    y = jnp.dot(der_ref[0], w, preferred_element_type=jnp.float32)
    out_ref[0, 0] = jnp.sum(y * y)


def kernel(x, der_desc_wrt_coord):
    B, A, D, K = der_desc_wrt_coord.shape
    der2 = der_desc_wrt_coord.reshape(B, A, D * K)
    x3 = jnp.repeat(x, K, axis=1)
    out = pl.pallas_call(
        _body,
        grid=(B,),
        in_specs=[
            pl.BlockSpec((1, D * K), lambda b: (b, 0)),
            pl.BlockSpec((1, A, D * K), lambda b: (b, 0, 0)),
        ],
        out_specs=pl.BlockSpec((1, 1), lambda b: (b, 0)),
        out_shape=jax.ShapeDtypeStruct((B, 1), jnp.float32),
    )(x3, der2)
    return out[:, 0]


# speedup vs baseline: 1044.0110x; 1044.0110x over previous
import jax
import jax.numpy as jnp
from jax import lax
from jax.experimental import pallas as pl


def _body(x3_ref, der_ref, out_ref):
    n = x3_ref.shape[2]
    j3 = lax.broadcasted_iota(jnp.int32, (n, 3), 0) % 3
    k3 = lax.broadcasted_iota(jnp.int32, (n, 3), 1)
    w = jnp.where(j3 == k3, x3_ref[0, 0][:, None], 0.0)
    y = jnp.dot(der_ref[0], w, preferred_element_type=jnp.float32)
    out_ref[...] = jnp.sum(y * y, keepdims=True)[None]


def kernel(x, der_desc_wrt_coord):
    B, A, D, K = der_desc_wrt_coord.shape
    der2 = der_desc_wrt_coord.reshape(B, A, D * K)
    x3 = jnp.repeat(x, K, axis=1).reshape(B, 1, D * K)
    out = pl.pallas_call(
        _body,
        grid=(B,),
        in_specs=[
            pl.BlockSpec((1, 1, D * K), lambda b: (b, 0, 0)),
            pl.BlockSpec((1, A, D * K), lambda b: (b, 0, 0)),
        ],
        out_specs=pl.BlockSpec((1, 1, 1), lambda b: (b, 0, 0)),
        out_shape=jax.ShapeDtypeStruct((B, 1, 1), jnp.float32),
    )(x3, der2)
    return out[:, 0, 0]
